# Initial kernel scaffold; baseline (speedup 1.0000x reference)
import jax
import jax.numpy as jnp
from jax.experimental import pallas as pl


def kernel(points, cam_params):
    raise NotImplementedError("write your pallas kernel here")


# speedup vs baseline: 1.0340x; 1.0340x over previous
import functools

import jax
import jax.numpy as jnp
from jax.experimental import pallas as pl
from jax.experimental.pallas import tpu as pltpu

S = 2048
SENT = 4 * S * S
_A = 6378137.0
_E2 = 6.69437999014e-3

_LANES = 1024
_RBLK = 8


def _transform_body(scal_ref, pts_ref, key_ref, *, n_valid, rows_per_blk):
    lat = pts_ref[0]
    lon = pts_ref[1]
    alt = pts_ref[2]

    x0 = scal_ref[0]
    y0 = scal_ref[1]
    z0 = scal_ref[2]
    neg_so = scal_ref[3]
    co = scal_ref[4]
    neg_sl_co = scal_ref[5]
    sl_so = scal_ref[6]
    cl = scal_ref[7]
    cl_co = scal_ref[8]
    cl_so = scal_ref[9]
    sl = scal_ref[10]
    r00 = scal_ref[11]
    r01 = scal_ref[12]
    r02 = scal_ref[13]
    r10 = scal_ref[14]
    r11 = scal_ref[15]
    r12 = scal_ref[16]
    r20 = scal_ref[17]
    r21 = scal_ref[18]
    r22 = scal_ref[19]

    latr = jnp.deg2rad(lat)
    lonr = jnp.deg2rad(lon)
    s = jnp.sin(latr)
    c = jnp.cos(latr)
    Nv = _A * jax.lax.rsqrt(1.0 - _E2 * s * s)
    X = (Nv + alt) * c * jnp.cos(lonr)
    Y = (Nv + alt) * c * jnp.sin(lonr)
    Z = (Nv * (1.0 - _E2) + alt) * s

    dx = X - x0
    dy = Y - y0
    dz = Z - z0
    e = neg_so * dx + co * dy
    n = neg_sl_co * dx - sl_so * dy + cl * dz
    u = cl_co * dx + cl_so * dy + sl * dz

    x = r00 * e + r01 * n + r02 * u
    y = r10 * e + r11 * n + r12 * u
    z = r20 * e + r21 * n + r22 * u

    ax = jnp.abs(x)
    ay = jnp.abs(y)
    az = jnp.abs(z)
    m_front = (z > 0) & (z > ax) & (z > ay)
    m_back = (z < 0) & (-z > ax) & (-z > ay)
    m_right = (x > 0) & (x > az) & (x > ay)
    m_left = (x < 0) & (-x > az) & (-x > ay)

    f = S / 2.0

    def cam2key(px_x, px_y, px_z):
        z_safe = jnp.where(jnp.abs(px_z) > 1e-9, px_z, 1.0)
        pu = f * px_x / z_safe + f
        pv = f * px_y / z_safe + f
        iu = jnp.clip(jnp.floor(pu), 0, S - 1).astype(jnp.int32)
        iv = jnp.clip(jnp.floor(pv), 0, S - 1).astype(jnp.int32)
        return iu * S + iv

    kf = cam2key(x, y, z)
    kb = cam2key(x, -y, z)
    kr = cam2key(-z, y, x)
    kl = cam2key(z, y, -x)

    key = jnp.where(
        m_front, kf,
        jnp.where(m_back, S * S + kb,
                  jnp.where(m_left, 2 * S * S + kl,
                            jnp.where(m_right, 3 * S * S + kr, SENT))))

    pid = pl.program_id(0)
    row = jax.lax.broadcasted_iota(jnp.int32, (rows_per_blk, _LANES), 0)
    col = jax.lax.broadcasted_iota(jnp.int32, (rows_per_blk, _LANES), 1)
    gidx = (pid * rows_per_blk + row) * _LANES + col
    key_ref[...] = jnp.where(gidx < n_valid, key, SENT)


def _compute_keys(points, cam_params):
    n = points.shape[0]
    rows = pl.cdiv(n, _RBLK * _LANES) * _RBLK
    n_pad = rows * _LANES
    grid = rows // _RBLK

    pts_t = jnp.transpose(points[:, :3])
    pts_t = jnp.pad(pts_t, ((0, 0), (0, n_pad - n)))
    pts_t = pts_t.reshape(3, rows, _LANES)

    lat0, lon0, alt0 = cam_params[0], cam_params[1], cam_params[2]
    latr0 = jnp.deg2rad(lat0)
    lonr0 = jnp.deg2rad(lon0)
    sl = jnp.sin(latr0)
    cl = jnp.cos(latr0)
    so = jnp.sin(lonr0)
    co = jnp.cos(lonr0)
    Nv0 = _A / jnp.sqrt(1.0 - _E2 * sl * sl)
    x0 = (Nv0 + alt0) * cl * jnp.cos(lonr0)
    y0 = (Nv0 + alt0) * cl * jnp.sin(lonr0)
    z0 = (Nv0 * (1.0 - _E2) + alt0) * sl

    qs = -cam_params[3]
    qx = cam_params[4]
    qy = cam_params[5]
    qz = cam_params[6]
    nrm = jnp.sqrt(qs * qs + qx * qx + qy * qy + qz * qz) + 1e-12
    qs, qx, qy, qz = qs / nrm, qx / nrm, qy / nrm, qz / nrm
    r00 = 1 - 2 * (qy * qy + qz * qz)
    r01 = 2 * (qx * qy - qz * qs)
    r02 = 2 * (qx * qz + qy * qs)
    r10 = 2 * (qx * qy + qz * qs)
    r11 = 1 - 2 * (qx * qx + qz * qz)
    r12 = 2 * (qy * qz - qx * qs)
    r20 = 2 * (qx * qz - qy * qs)
    r21 = 2 * (qy * qz + qx * qs)
    r22 = 1 - 2 * (qx * qx + qy * qy)

    scal = jnp.stack([
        x0, y0, z0, -so, co, -sl * co, sl * so, cl, cl * co, cl * so, sl,
        r00, r01, r02, r10, r11, r12, r20, r21, r22,
    ]).astype(jnp.float32)

    keys = pl.pallas_call(
        functools.partial(_transform_body, n_valid=n, rows_per_blk=_RBLK),
        grid=(grid,),
        in_specs=[
            pl.BlockSpec(memory_space=pltpu.SMEM),
            pl.BlockSpec((3, _RBLK, _LANES), lambda i: (0, i, 0)),
        ],
        out_specs=pl.BlockSpec((_RBLK, _LANES), lambda i: (i, 0)),
        out_shape=jax.ShapeDtypeStruct((rows, _LANES), jnp.int32),
    )(scal, pts_t)
    return keys.reshape(-1)[:n]


def kernel(points, cam_params):
    n = points.shape[0]
    keys = _compute_keys(points, cam_params)
    inten = points[:, 3]

    def face(i):
        off = i * S * S
        idx = jnp.where((keys >= off) & (keys < off + S * S), keys - off, S * S)
        buf = jnp.zeros((S * S + 8,), jnp.float32).at[idx].set(inten)
        return buf[:S * S].reshape(S, S)

    return face(0), face(1), face(2), face(3)
